# baseline (speedup 1.0000x reference)
import dataclasses
import functools

import jax
import jax.numpy as jnp
from jax import lax
from jax.experimental import pallas as pl
from jax.experimental.pallas import tpu as pltpu
from jax.experimental.pallas import tpu_sc as plsc

NBINS = 256
NBATCH = 16
NROWS = 4096
NC, NS, L = 2, 16, 16
HROWS = NROWS // 2
CROWS = 128
NCHUNK = HROWS // CROWS
NTAB = 4


def kernel(input, weight):
    mesh = plsc.VectorSubcoreMesh(
        core_axis_name="c", subcore_axis_name="s",
        num_cores=NC, num_subcores=NS)

    cp = pltpu.CompilerParams()
    if "needs_layout_passes" in pltpu.CompilerParams.__dataclass_fields__:
        cp = dataclasses.replace(cp, needs_layout_passes=False)

    @functools.partial(
        pl.kernel,
        mesh=mesh,
        compiler_params=cp,
        out_type=jax.ShapeDtypeStruct((NBATCH, NBINS), jnp.float32),
        scratch_types=[
            pltpu.VMEM((CROWS, NBINS), jnp.float32),
            pltpu.VMEM((CROWS, NBINS), jnp.float32),
            pltpu.VMEM((NBINS,), jnp.float32),
            pltpu.VMEM((NBINS,), jnp.float32),
            pltpu.VMEM((NBINS,), jnp.float32),
            pltpu.VMEM((NBINS,), jnp.float32),
            pltpu.VMEM((NBINS,), jnp.float32),
            pltpu.VMEM((NBINS,), jnp.float32),
            pltpu.VMEM((NBINS,), jnp.float32),
            pltpu.VMEM((NBINS,), jnp.float32),
            pltpu.VMEM((NBINS,), jnp.float32),
            pltpu.VMEM_SHARED((NS, NBINS), jnp.float32),
            pltpu.SemaphoreType.DMA,
            pltpu.SemaphoreType.DMA,
        ],
    )
    def hist_kernel(x_hbm, w_hbm, out_hbm, buf0, buf1, ht0, ht1, ht2, ht3,
                    hist, wloc, tmp0, tmp1, outv, shared, sem0, sem1):
        htabs = (ht0, ht1, ht2, ht3)
        c = lax.axis_index("c")
        s = lax.axis_index("s")
        b = 8 * c + s // 2
        h = s % 2
        row0 = h * HROWS

        def src(k):
            return x_hbm.at[b, pl.ds(row0 + k * CROWS, CROWS), :]

        zeros16 = jnp.zeros((L,), jnp.float32)
        ones16 = jnp.ones((L,), jnp.float32)

        @pl.loop(0, NBINS, step=L)
        def _(i):
            for t in range(NTAB):
                htabs[t][pl.ds(i, L)] = zeros16

        pltpu.sync_copy(w_hbm, wloc)

        def process(bref):
            def cvt_row(r):
                return [(bref[r, pl.ds(u * L, L)]
                         * jnp.float32(NBINS)).astype(jnp.int32)
                        for u in range(NBINS // L)]

            def scat(ids):
                for u in range(NBINS // L):
                    plsc.addupdate_scatter(htabs[u % NTAB], [ids[u]], ones16)

            @plsc.parallel_loop(1, CROWS, carry=cvt_row(0))
            def fin(r, ids):
                new_ids = cvt_row(r)
                scat(ids)
                return new_ids

            scat(fin)

        pltpu.async_copy(src(0), buf0, sem0)

        @pl.loop(0, NCHUNK, step=2)
        def _(k):
            pltpu.async_copy(src(k + 1), buf1, sem1)
            pltpu.make_async_copy(src(k), buf0, sem0).wait()
            process(buf0)

            @pl.when(k + 2 < NCHUNK)
            def _():
                pltpu.async_copy(src(k + 2), buf0, sem0)

            pltpu.make_async_copy(src(k + 1), buf1, sem1).wait()
            process(buf1)

        @pl.loop(0, NBINS, step=L)
        def _(i):
            acc = htabs[0][pl.ds(i, L)]
            for t in range(1, NTAB):
                acc = acc + htabs[t][pl.ds(i, L)]
            hist[pl.ds(i, L)] = acc

        pltpu.sync_copy(hist, shared.at[s])
        plsc.subcore_barrier()

        @pl.when(s < 8)
        def _():
            pltpu.sync_copy(shared.at[2 * s], tmp0)
            pltpu.sync_copy(shared.at[2 * s + 1], tmp1)

            @pl.loop(0, NBINS, step=L)
            def _(i):
                slc = pl.ds(i, L)
                outv[slc] = (tmp0[slc] + tmp1[slc]) * wloc[slc]

            pltpu.sync_copy(outv, out_hbm.at[8 * c + s])

    return hist_kernel(input, weight)


# speedup vs baseline: 1.2650x; 1.2650x over previous
import dataclasses
import functools

import jax
import jax.numpy as jnp
from jax import lax
from jax.experimental import pallas as pl
from jax.experimental.pallas import tpu as pltpu
from jax.experimental.pallas import tpu_sc as plsc

NBINS = 256
NBATCH = 16
NROWS = 4096
NC, NS, L = 2, 16, 16
HROWS = NROWS // 2
CROWS = 128
NCHUNK = HROWS // CROWS
NTAB = 4


def kernel(input, weight):
    mesh = plsc.VectorSubcoreMesh(
        core_axis_name="c", subcore_axis_name="s",
        num_cores=NC, num_subcores=NS)

    cp = pltpu.CompilerParams()
    if "needs_layout_passes" in pltpu.CompilerParams.__dataclass_fields__:
        cp = dataclasses.replace(cp, needs_layout_passes=False)

    @functools.partial(
        pl.kernel,
        mesh=mesh,
        compiler_params=cp,
        out_type=jax.ShapeDtypeStruct((NBATCH, NBINS), jnp.float32),
        scratch_types=[
            pltpu.VMEM((CROWS, NBINS), jnp.float32),
            pltpu.VMEM((CROWS, NBINS), jnp.float32),
            pltpu.VMEM((NBINS,), jnp.float32),
            pltpu.VMEM((NBINS,), jnp.float32),
            pltpu.VMEM((NBINS,), jnp.float32),
            pltpu.VMEM((NBINS,), jnp.float32),
            pltpu.VMEM((NBINS,), jnp.float32),
            pltpu.VMEM((NBINS,), jnp.float32),
            pltpu.VMEM((NBINS,), jnp.float32),
            pltpu.VMEM((NBINS,), jnp.float32),
            pltpu.VMEM((NBINS,), jnp.float32),
            pltpu.VMEM_SHARED((NS, NBINS), jnp.float32),
            pltpu.SemaphoreType.DMA,
            pltpu.SemaphoreType.DMA,
        ],
    )
    def hist_kernel(x_hbm, w_hbm, out_hbm, buf0, buf1, ht0, ht1, ht2, ht3,
                    hist, wloc, tmp0, tmp1, outv, shared, sem0, sem1):
        htabs = (ht0, ht1, ht2, ht3)
        c = lax.axis_index("c")
        s = lax.axis_index("s")
        b = 8 * c + s // 2
        h = s % 2
        row0 = h * HROWS

        def src(k):
            return x_hbm.at[b, pl.ds(row0 + k * CROWS, CROWS), :]

        zeros16 = jnp.zeros((L,), jnp.float32)
        ones16 = jnp.ones((L,), jnp.float32)

        @pl.loop(0, NBINS, step=L)
        def _(i):
            for t in range(NTAB):
                htabs[t][pl.ds(i, L)] = zeros16

        pltpu.sync_copy(w_hbm, wloc)

        def process(bref):
            @plsc.parallel_loop(0, CROWS, unroll=4)
            def _(r):
                xs = [bref[r, pl.ds(u * L, L)] for u in range(NBINS // L)]
                ids = [(xv * jnp.float32(NBINS)).astype(jnp.int32)
                       for xv in xs]
                for u in range(NBINS // L):
                    plsc.addupdate_scatter(htabs[u % NTAB], [ids[u]], ones16)

        pltpu.async_copy(src(0), buf0, sem0)

        @pl.loop(0, NCHUNK, step=2)
        def _(k):
            pltpu.async_copy(src(k + 1), buf1, sem1)
            pltpu.make_async_copy(src(k), buf0, sem0).wait()
            process(buf0)

            @pl.when(k + 2 < NCHUNK)
            def _():
                pltpu.async_copy(src(k + 2), buf0, sem0)

            pltpu.make_async_copy(src(k + 1), buf1, sem1).wait()
            process(buf1)

        @pl.loop(0, NBINS, step=L)
        def _(i):
            acc = htabs[0][pl.ds(i, L)]
            for t in range(1, NTAB):
                acc = acc + htabs[t][pl.ds(i, L)]
            hist[pl.ds(i, L)] = acc

        pltpu.sync_copy(hist, shared.at[s])
        plsc.subcore_barrier()

        @pl.when(s < 8)
        def _():
            pltpu.sync_copy(shared.at[2 * s], tmp0)
            pltpu.sync_copy(shared.at[2 * s + 1], tmp1)

            @pl.loop(0, NBINS, step=L)
            def _(i):
                slc = pl.ds(i, L)
                outv[slc] = (tmp0[slc] + tmp1[slc]) * wloc[slc]

            pltpu.sync_copy(outv, out_hbm.at[8 * c + s])

    return hist_kernel(input, weight)
